# Initial kernel scaffold; baseline (speedup 1.0000x reference)
import jax
import jax.numpy as jnp
from jax.experimental import pallas as pl


def kernel(x, emb_table, W, b):
    raise NotImplementedError("write your pallas kernel here")


# speedup vs baseline: 306.9061x; 306.9061x over previous
import jax
import jax.numpy as jnp
from jax import lax
from jax.experimental import pallas as pl
from jax.experimental.pallas import tpu as pltpu
from jax.experimental.pallas import tpu_sc as plsc

FIELD0 = 1000
NUM_IN = 2000
BATCH = 1024
NC = 2
NS = 16
NW = NC * NS
BPW = BATCH // NW
L = 16


def _fm_body(u_hbm, m_hbm, r_hbm, emb_hbm, w_hbm, b_hbm,
             out_hbm, rec_hbm,
             u_v, m_v, r_v, mi_v, wu_v, wm_v, e_v, b_v, o_v, rec_v, red_v,
             sem_u, sem_m):
    wid = lax.axis_index("s") * NC + lax.axis_index("c")
    base = wid * BPW

    pltpu.sync_copy(u_hbm.at[pl.ds(base, BPW)], u_v)
    pltpu.sync_copy(m_hbm.at[pl.ds(base, BPW)], m_v)
    pltpu.sync_copy(r_hbm.at[pl.ds(base, BPW)], r_v)
    pltpu.sync_copy(emb_hbm.at[pl.ds(0, 2)], e_v)
    pltpu.sync_copy(b_hbm, b_v)

    for j in range(BPW // L):
        sl = pl.ds(j * L, L)
        mi_v[sl] = m_v[sl] + FIELD0

    cp_u = pltpu.async_copy(w_hbm.at[u_v], wu_v, sem_u)
    cp_m = pltpu.async_copy(w_hbm.at[mi_v], wm_v, sem_m)

    t0 = e_v[0, :]
    t1 = e_v[1, :]
    s = (NUM_IN - 2.0) * t0 + 2.0 * t1
    sq = s * s - ((NUM_IN - 2.0) * t0 * t0 + 2.0 * t1 * t1)
    lanes = lax.iota(jnp.int32, L)
    acc = sq
    for stride in (8, 4, 2, 1):
        red_v[...] = acc
        acc = acc + plsc.load_gather(red_v, [lanes ^ stride])
    cb = b_v[...] + 0.5 * acc

    cp_u.wait()
    cp_m.wait()

    for j in range(BPW // L):
        sl = pl.ds(j * L, L)
        z = wu_v[sl] + wm_v[sl] + cb
        o_v[sl] = 1.0 / (1.0 + jnp.exp(-z))
        rec_v[sl] = jnp.where(r_v[sl] >= 3, 1.0, 0.0)

    pltpu.sync_copy(o_v, out_hbm.at[pl.ds(base, BPW)])
    pltpu.sync_copy(rec_v, rec_hbm.at[pl.ds(base, BPW)])


def kernel(x, emb_table, W, b):
    x32 = x.astype(jnp.int32)
    u = x32[:, 0]
    m = x32[:, 1]
    r = x32[:, 2]
    w_flat = W.reshape(NUM_IN)
    b_vec = jnp.broadcast_to(b.astype(jnp.float32), (L,))

    mesh = plsc.VectorSubcoreMesh(core_axis_name="c", subcore_axis_name="s")
    out, rec = pl.kernel(
        _fm_body,
        mesh=mesh,
        out_type=[jax.ShapeDtypeStruct((BATCH,), jnp.float32),
                  jax.ShapeDtypeStruct((BATCH,), jnp.float32)],
        scratch_types=[
            pltpu.VMEM((BPW,), jnp.int32),
            pltpu.VMEM((BPW,), jnp.int32),
            pltpu.VMEM((BPW,), jnp.int32),
            pltpu.VMEM((BPW,), jnp.int32),
            pltpu.VMEM((BPW,), jnp.float32),
            pltpu.VMEM((BPW,), jnp.float32),
            pltpu.VMEM((2, L), jnp.float32),
            pltpu.VMEM((L,), jnp.float32),
            pltpu.VMEM((BPW,), jnp.float32),
            pltpu.VMEM((BPW,), jnp.float32),
            pltpu.VMEM((L,), jnp.float32),
            pltpu.SemaphoreType.DMA,
            pltpu.SemaphoreType.DMA,
        ],
        compiler_params=pltpu.CompilerParams(needs_layout_passes=False),
    )(u, m, r, emb_table, w_flat, b_vec)
    return out.reshape(BATCH, 1), rec.reshape(BATCH, 1)


# speedup vs baseline: 334.8039x; 1.0909x over previous
import jax
import jax.numpy as jnp
from jax import lax
from jax.experimental import pallas as pl
from jax.experimental.pallas import tpu as pltpu
from jax.experimental.pallas import tpu_sc as plsc

FIELD0 = 1000
NUM_IN = 2000
BATCH = 1024
NC = 2
NS = 16
NW = NC * NS
BPW = BATCH // NW
L = 16


def _fm_body(u_hbm, m_hbm, r_hbm, emb_hbm, w_hbm, b_hbm,
             out_hbm, rec_hbm,
             u_v, m_v, r_v, mi_v, wu_v, wm_v, e_v, b_v, o_v, rec_v, red_v,
             sem_in, sem_u, sem_m):
    wid = lax.axis_index("s") * NC + lax.axis_index("c")
    base = wid * BPW

    cp_um = pltpu.async_copy(u_hbm.at[pl.ds(base, BPW)], u_v, sem_in)
    cp_m = pltpu.async_copy(m_hbm.at[pl.ds(base, BPW)], m_v, sem_in)
    cp_r = pltpu.async_copy(r_hbm.at[pl.ds(base, BPW)], r_v, sem_in)
    cp_e = pltpu.async_copy(emb_hbm.at[pl.ds(0, 2)], e_v, sem_in)
    cp_b = pltpu.async_copy(b_hbm, b_v, sem_in)
    cp_um.wait()
    cp_m.wait()

    for j in range(BPW // L):
        sl = pl.ds(j * L, L)
        mi_v[sl] = m_v[sl] + FIELD0

    cp_u = pltpu.async_copy(w_hbm.at[u_v], wu_v, sem_u)
    cp_w = pltpu.async_copy(w_hbm.at[mi_v], wm_v, sem_m)

    cp_r.wait()
    cp_e.wait()
    cp_b.wait()

    t0 = e_v[0, :]
    t1 = e_v[1, :]
    s = (NUM_IN - 2.0) * t0 + 2.0 * t1
    sq = s * s - ((NUM_IN - 2.0) * t0 * t0 + 2.0 * t1 * t1)
    lanes = lax.iota(jnp.int32, L)
    acc = sq
    for stride in (8, 4, 2, 1):
        red_v[...] = acc
        acc = acc + plsc.load_gather(red_v, [lanes ^ stride])
    cb = b_v[...] + 0.5 * acc

    cp_u.wait()
    cp_w.wait()

    for j in range(BPW // L):
        sl = pl.ds(j * L, L)
        z = wu_v[sl] + wm_v[sl] + cb
        o_v[sl] = 1.0 / (1.0 + jnp.exp(-z))
        rec_v[sl] = jnp.where(r_v[sl] >= 3, 1.0, 0.0)

    cp_o = pltpu.async_copy(o_v, out_hbm.at[pl.ds(base, BPW)], sem_u)
    cp_rec = pltpu.async_copy(rec_v, rec_hbm.at[pl.ds(base, BPW)], sem_m)
    cp_o.wait()
    cp_rec.wait()


def kernel(x, emb_table, W, b):
    x32 = x.astype(jnp.int32)
    u = x32[:, 0]
    m = x32[:, 1]
    r = x32[:, 2]
    w_flat = W.reshape(NUM_IN)
    b_vec = jnp.broadcast_to(b.astype(jnp.float32), (L,))

    mesh = plsc.VectorSubcoreMesh(core_axis_name="c", subcore_axis_name="s")
    out, rec = pl.kernel(
        _fm_body,
        mesh=mesh,
        out_type=[jax.ShapeDtypeStruct((BATCH,), jnp.float32),
                  jax.ShapeDtypeStruct((BATCH,), jnp.float32)],
        scratch_types=[
            pltpu.VMEM((BPW,), jnp.int32),
            pltpu.VMEM((BPW,), jnp.int32),
            pltpu.VMEM((BPW,), jnp.int32),
            pltpu.VMEM((BPW,), jnp.int32),
            pltpu.VMEM((BPW,), jnp.float32),
            pltpu.VMEM((BPW,), jnp.float32),
            pltpu.VMEM((2, L), jnp.float32),
            pltpu.VMEM((L,), jnp.float32),
            pltpu.VMEM((BPW,), jnp.float32),
            pltpu.VMEM((BPW,), jnp.float32),
            pltpu.VMEM((L,), jnp.float32),
            pltpu.SemaphoreType.DMA,
            pltpu.SemaphoreType.DMA,
            pltpu.SemaphoreType.DMA,
        ],
        compiler_params=pltpu.CompilerParams(needs_layout_passes=False),
    )(u, m, r, emb_table, w_flat, b_vec)
    return out.reshape(BATCH, 1), rec.reshape(BATCH, 1)
